# baseline (speedup 1.0000x reference)
import functools
import jax
import jax.numpy as jnp
from jax import lax
from jax.experimental import pallas as pl
from jax.experimental.pallas import tpu as pltpu


def _compactor_body(k_ref, v_ref, a_ref, ko_ref, vo_ref, z_ref, *, n_t, d):
    t = pl.program_id(0)
    g = pl.program_id(1)
    k = k_ref[:, pl.ds(g * d, d)]
    v = v_ref[:, pl.ds(g * d, d)]
    a = a_ref[g]
    TB = k.shape[0]
    M = a.shape[0]

    scores = lax.dot_general(k, a, (((1,), (1,)), ((), ())),
                             preferred_element_type=jnp.float32)
    mx = jnp.max(scores, axis=1, keepdims=True)
    negidx = lax.broadcasted_iota(jnp.int32, (TB, M), 1).astype(jnp.float32) * -1.0
    cand = jnp.where(scores == mx, negidx, -jnp.inf)
    topneg = jnp.max(cand, axis=1, keepdims=True)
    onehot = (negidx == topneg).astype(jnp.float32)

    ck = lax.dot_general(onehot, k, (((0,), (0,)), ((), ())),
                         preferred_element_type=jnp.float32)
    cv = lax.dot_general(onehot, v, (((0,), (0,)), ((), ())),
                         preferred_element_type=jnp.float32)
    z = jnp.sum(onehot, axis=0)[None, :]

    @pl.when(t == 0)
    def _init():
        ko_ref[g] = ck
        vo_ref[g] = cv
        z_ref[g, :] = z[0]

    @pl.when(t > 0)
    def _acc():
        ko_ref[g] += ck
        vo_ref[g] += cv
        z_ref[g, :] += z[0]

    @pl.when(t == n_t - 1)
    def _norm():
        zs = jnp.clip(z_ref[g, :], 1e-8, None)[:, None]
        ko_ref[g] = ko_ref[g] / zs
        vo_ref[g] = vo_ref[g] / zs


def kernel(K_cold, V_cold, anchors):
    T, H, D = K_cold.shape
    G, M, _ = anchors.shape
    TB = min(256, T)
    n_t = T // TB

    Kf = K_cold.reshape(T, H * D)
    Vf = V_cold.reshape(T, H * D)

    grid = (n_t, G)
    out_shape = [
        jax.ShapeDtypeStruct((G, M, D), jnp.float32),
        jax.ShapeDtypeStruct((G, M, D), jnp.float32),
    ]
    k_acc, v_acc = pl.pallas_call(
        functools.partial(_compactor_body, n_t=n_t, d=D),
        grid=grid,
        in_specs=[
            pl.BlockSpec((TB, H * D), lambda t, g: (t, 0)),
            pl.BlockSpec((TB, H * D), lambda t, g: (t, 0)),
            pl.BlockSpec((G, M, D), lambda t, g: (0, 0, 0)),
        ],
        out_specs=[
            pl.BlockSpec((G, M, D), lambda t, g: (0, 0, 0)),
            pl.BlockSpec((G, M, D), lambda t, g: (0, 0, 0)),
        ],
        scratch_shapes=[pltpu.VMEM((G, M), jnp.float32)],
        out_shape=out_shape,
    )(Kf, Vf, anchors)

    K_mem = jnp.transpose(k_acc, (1, 0, 2)).astype(K_cold.dtype)
    V_mem = jnp.transpose(v_acc, (1, 0, 2)).astype(V_cold.dtype)
    return (K_mem, V_mem)


# speedup vs baseline: 2.1419x; 2.1419x over previous
import functools
import jax
import jax.numpy as jnp
from jax import lax
from jax.experimental import pallas as pl
from jax.experimental.pallas import tpu as pltpu


def _compactor_body(k_ref, v_ref, a_ref, ko_ref, vo_ref, z_ref, *, n_t):
    t = pl.program_id(1)
    k = k_ref[0]
    v = v_ref[0]
    a = a_ref[0]
    TB, D = k.shape
    M = a.shape[0]

    scores = lax.dot_general(k, a, (((1,), (1,)), ((), ())),
                             preferred_element_type=jnp.float32)
    mx = jnp.max(scores, axis=1, keepdims=True)
    negidx = lax.broadcasted_iota(jnp.int32, (TB, M), 1).astype(jnp.float32) * -1.0
    cand = jnp.where(scores == mx, negidx, -jnp.inf)
    topneg = jnp.max(cand, axis=1, keepdims=True)
    onehot = (negidx == topneg).astype(jnp.float32)

    ck = lax.dot_general(onehot, k, (((0,), (0,)), ((), ())),
                         preferred_element_type=jnp.float32)
    cv = lax.dot_general(onehot, v, (((0,), (0,)), ((), ())),
                         preferred_element_type=jnp.float32)
    z = jnp.sum(onehot, axis=0)[None, :]

    @pl.when(t == 0)
    def _init():
        ko_ref[0] = ck
        vo_ref[0] = cv
        z_ref[...] = z

    @pl.when(t > 0)
    def _acc():
        ko_ref[0] += ck
        vo_ref[0] += cv
        z_ref[...] += z

    @pl.when(t == n_t - 1)
    def _norm():
        zs = jnp.clip(z_ref[...], 1e-8, None)[0, :, None]
        ko_ref[0] = ko_ref[0] / zs
        vo_ref[0] = vo_ref[0] / zs


def kernel(K_cold, V_cold, anchors):
    T, H, D = K_cold.shape
    G, M, _ = anchors.shape
    TB = min(1024, T)
    n_t = T // TB

    Kg = jnp.transpose(K_cold, (1, 0, 2))
    Vg = jnp.transpose(V_cold, (1, 0, 2))

    grid = (G, n_t)
    out_shape = [
        jax.ShapeDtypeStruct((G, M, D), jnp.float32),
        jax.ShapeDtypeStruct((G, M, D), jnp.float32),
    ]
    k_acc, v_acc = pl.pallas_call(
        functools.partial(_compactor_body, n_t=n_t),
        grid=grid,
        in_specs=[
            pl.BlockSpec((1, TB, D), lambda g, t: (g, t, 0)),
            pl.BlockSpec((1, TB, D), lambda g, t: (g, t, 0)),
            pl.BlockSpec((1, M, D), lambda g, t: (g, 0, 0)),
        ],
        out_specs=[
            pl.BlockSpec((1, M, D), lambda g, t: (g, 0, 0)),
            pl.BlockSpec((1, M, D), lambda g, t: (g, 0, 0)),
        ],
        scratch_shapes=[pltpu.VMEM((1, M), jnp.float32)],
        out_shape=out_shape,
    )(Kg, Vg, anchors)

    K_mem = jnp.transpose(k_acc, (1, 0, 2)).astype(K_cold.dtype)
    V_mem = jnp.transpose(v_acc, (1, 0, 2)).astype(V_cold.dtype)
    return (K_mem, V_mem)


# speedup vs baseline: 2.5333x; 1.1827x over previous
import functools
import jax
import jax.numpy as jnp
from jax import lax
from jax.experimental import pallas as pl
from jax.experimental.pallas import tpu as pltpu


def _compactor_body(k_ref, v_ref, a_ref, ko_ref, vo_ref, z_ref, *, n_t):
    t = pl.program_id(1)
    k = k_ref[0]
    v = v_ref[0]
    a = a_ref[0]
    TB, D = k.shape
    M = a.shape[0]

    scores = lax.dot_general(k, a, (((1,), (1,)), ((), ())),
                             preferred_element_type=jnp.float32)
    mx = jnp.max(scores, axis=1, keepdims=True)
    negidx = lax.broadcasted_iota(jnp.int32, (TB, M), 1).astype(jnp.float32) * -1.0
    cand = jnp.where(scores == mx, negidx, -jnp.inf)
    topneg = jnp.max(cand, axis=1, keepdims=True)
    onehot = (negidx == topneg).astype(jnp.float32)

    ck = lax.dot_general(onehot, k, (((0,), (0,)), ((), ())),
                         preferred_element_type=jnp.float32)
    cv = lax.dot_general(onehot, v, (((0,), (0,)), ((), ())),
                         preferred_element_type=jnp.float32)
    z = jnp.sum(onehot, axis=0)[None, :]

    @pl.when(t == 0)
    def _init():
        ko_ref[0] = ck
        vo_ref[0] = cv
        z_ref[...] = z

    @pl.when(t > 0)
    def _acc():
        ko_ref[0] += ck
        vo_ref[0] += cv
        z_ref[...] += z

    @pl.when(t == n_t - 1)
    def _norm():
        zs = jnp.clip(z_ref[...], 1e-8, None)[0, :, None]
        ko_ref[0] = ko_ref[0] / zs
        vo_ref[0] = vo_ref[0] / zs


def kernel(K_cold, V_cold, anchors):
    T, H, D = K_cold.shape
    G, M, _ = anchors.shape
    TB = min(2048, T)
    n_t = T // TB

    Kg = jnp.transpose(K_cold, (1, 0, 2))
    Vg = jnp.transpose(V_cold, (1, 0, 2))

    grid = (G, n_t)
    out_shape = [
        jax.ShapeDtypeStruct((G, M, D), jnp.float32),
        jax.ShapeDtypeStruct((G, M, D), jnp.float32),
    ]
    k_acc, v_acc = pl.pallas_call(
        functools.partial(_compactor_body, n_t=n_t),
        grid=grid,
        in_specs=[
            pl.BlockSpec((1, TB, D), lambda g, t: (g, t, 0)),
            pl.BlockSpec((1, TB, D), lambda g, t: (g, t, 0)),
            pl.BlockSpec((1, M, D), lambda g, t: (g, 0, 0)),
        ],
        out_specs=[
            pl.BlockSpec((1, M, D), lambda g, t: (g, 0, 0)),
            pl.BlockSpec((1, M, D), lambda g, t: (g, 0, 0)),
        ],
        scratch_shapes=[pltpu.VMEM((1, M), jnp.float32)],
        out_shape=out_shape,
    )(Kg, Vg, anchors)

    K_mem = jnp.transpose(k_acc, (1, 0, 2)).astype(K_cold.dtype)
    V_mem = jnp.transpose(v_acc, (1, 0, 2)).astype(V_cold.dtype)
    return (K_mem, V_mem)


# speedup vs baseline: 2.6811x; 1.0584x over previous
import functools
import jax
import jax.numpy as jnp
from jax import lax
from jax.experimental import pallas as pl
from jax.experimental.pallas import tpu as pltpu


def _compactor_body(k_ref, v_ref, a_ref, ko_ref, vo_ref, z_ref, *, n_t):
    t = pl.program_id(1)
    k = k_ref[0]
    v = v_ref[0]
    a = a_ref[0]
    TB, D = k.shape
    M = a.shape[0]

    scores = lax.dot_general(k, a, (((1,), (1,)), ((), ())),
                             preferred_element_type=jnp.float32)
    mx = jnp.max(scores, axis=1, keepdims=True)
    negidx = lax.broadcasted_iota(jnp.int32, (TB, M), 1).astype(jnp.float32) * -1.0
    cand = jnp.where(scores == mx, negidx, -jnp.inf)
    topneg = jnp.max(cand, axis=1, keepdims=True)
    onehot = (negidx == topneg).astype(jnp.float32)

    ck = lax.dot_general(onehot, k, (((0,), (0,)), ((), ())),
                         preferred_element_type=jnp.float32)
    cv = lax.dot_general(onehot, v, (((0,), (0,)), ((), ())),
                         preferred_element_type=jnp.float32)
    z = jnp.sum(onehot, axis=0)[None, :]

    @pl.when(t == 0)
    def _init():
        ko_ref[0] = ck
        vo_ref[0] = cv
        z_ref[...] = z

    @pl.when(t > 0)
    def _acc():
        ko_ref[0] += ck
        vo_ref[0] += cv
        z_ref[...] += z

    @pl.when(t == n_t - 1)
    def _norm():
        zs = jnp.clip(z_ref[...], 1e-8, None)[0, :, None]
        ko_ref[0] = ko_ref[0] / zs
        vo_ref[0] = vo_ref[0] / zs


def kernel(K_cold, V_cold, anchors):
    T, H, D = K_cold.shape
    G, M, _ = anchors.shape
    TB = min(4096, T)
    n_t = T // TB

    Kg = jnp.transpose(K_cold, (1, 0, 2))
    Vg = jnp.transpose(V_cold, (1, 0, 2))

    grid = (G, n_t)
    out_shape = [
        jax.ShapeDtypeStruct((G, M, D), jnp.float32),
        jax.ShapeDtypeStruct((G, M, D), jnp.float32),
    ]
    k_acc, v_acc = pl.pallas_call(
        functools.partial(_compactor_body, n_t=n_t),
        grid=grid,
        in_specs=[
            pl.BlockSpec((1, TB, D), lambda g, t: (g, t, 0)),
            pl.BlockSpec((1, TB, D), lambda g, t: (g, t, 0)),
            pl.BlockSpec((1, M, D), lambda g, t: (g, 0, 0)),
        ],
        out_specs=[
            pl.BlockSpec((1, M, D), lambda g, t: (g, 0, 0)),
            pl.BlockSpec((1, M, D), lambda g, t: (g, 0, 0)),
        ],
        scratch_shapes=[pltpu.VMEM((1, M), jnp.float32)],
        out_shape=out_shape,
    )(Kg, Vg, anchors)

    K_mem = jnp.transpose(k_acc, (1, 0, 2)).astype(K_cold.dtype)
    V_mem = jnp.transpose(v_acc, (1, 0, 2)).astype(V_cold.dtype)
    return (K_mem, V_mem)


# speedup vs baseline: 2.7804x; 1.0370x over previous
import functools
import jax
import jax.numpy as jnp
from jax import lax
from jax.experimental import pallas as pl
from jax.experimental.pallas import tpu as pltpu


def _compactor_body(k_ref, v_ref, a_ref, ko_ref, vo_ref, z_ref, *, n_t):
    t = pl.program_id(1)
    k = k_ref[0]
    v = v_ref[0]
    a = a_ref[0]
    TB, D = k.shape
    M = a.shape[0]

    scores = lax.dot_general(k, a, (((1,), (1,)), ((), ())),
                             preferred_element_type=jnp.float32)
    mx = jnp.max(scores, axis=1, keepdims=True)
    negidx = lax.broadcasted_iota(jnp.int32, (TB, M), 1).astype(jnp.float32) * -1.0
    cand = jnp.where(scores == mx, negidx, -jnp.inf)
    topneg = jnp.max(cand, axis=1, keepdims=True)
    onehot = (negidx == topneg).astype(jnp.float32)

    ck = lax.dot_general(onehot, k, (((0,), (0,)), ((), ())),
                         preferred_element_type=jnp.float32)
    cv = lax.dot_general(onehot, v, (((0,), (0,)), ((), ())),
                         preferred_element_type=jnp.float32)
    z = jnp.sum(onehot, axis=0)[None, :]

    @pl.when(t == 0)
    def _init():
        ko_ref[0] = ck
        vo_ref[0] = cv
        z_ref[...] = z

    @pl.when(t > 0)
    def _acc():
        ko_ref[0] += ck
        vo_ref[0] += cv
        z_ref[...] += z

    @pl.when(t == n_t - 1)
    def _norm():
        zs = jnp.clip(z_ref[...], 1e-8, None)[0, :, None]
        ko_ref[0] = ko_ref[0] / zs
        vo_ref[0] = vo_ref[0] / zs


def kernel(K_cold, V_cold, anchors):
    T, H, D = K_cold.shape
    G, M, _ = anchors.shape
    TB = min(8192, T)
    n_t = T // TB

    Kg = jnp.transpose(K_cold, (1, 0, 2))
    Vg = jnp.transpose(V_cold, (1, 0, 2))

    grid = (G, n_t)
    out_shape = [
        jax.ShapeDtypeStruct((G, M, D), jnp.float32),
        jax.ShapeDtypeStruct((G, M, D), jnp.float32),
    ]
    k_acc, v_acc = pl.pallas_call(
        functools.partial(_compactor_body, n_t=n_t),
        grid=grid,
        in_specs=[
            pl.BlockSpec((1, TB, D), lambda g, t: (g, t, 0)),
            pl.BlockSpec((1, TB, D), lambda g, t: (g, t, 0)),
            pl.BlockSpec((1, M, D), lambda g, t: (g, 0, 0)),
        ],
        out_specs=[
            pl.BlockSpec((1, M, D), lambda g, t: (g, 0, 0)),
            pl.BlockSpec((1, M, D), lambda g, t: (g, 0, 0)),
        ],
        scratch_shapes=[pltpu.VMEM((1, M), jnp.float32)],
        out_shape=out_shape,
    )(Kg, Vg, anchors)

    K_mem = jnp.transpose(k_acc, (1, 0, 2)).astype(K_cold.dtype)
    V_mem = jnp.transpose(v_acc, (1, 0, 2)).astype(V_cold.dtype)
    return (K_mem, V_mem)


# speedup vs baseline: 2.9198x; 1.0501x over previous
import functools
import jax
import jax.numpy as jnp
from jax import lax
from jax.experimental import pallas as pl
from jax.experimental.pallas import tpu as pltpu


def _compactor_body(k_ref, v_ref, a_ref, ko_ref, vo_ref, z_ref, *, n_t):
    t = pl.program_id(1)
    k = k_ref[0]
    v = v_ref[0]
    a = a_ref[0]
    TB, D = k.shape
    M = a.shape[0]

    scores = lax.dot_general(k, a, (((1,), (1,)), ((), ())),
                             preferred_element_type=jnp.float32)
    mx = jnp.max(scores, axis=1, keepdims=True)
    negidx = lax.broadcasted_iota(jnp.int32, (TB, M), 1).astype(jnp.float32) * -1.0
    cand = jnp.where(scores == mx, negidx, -jnp.inf)
    topneg = jnp.max(cand, axis=1, keepdims=True)
    onehot = (negidx == topneg).astype(jnp.float32)

    oh16 = onehot.astype(jnp.bfloat16)
    ck = lax.dot_general(oh16, k.astype(jnp.bfloat16), (((0,), (0,)), ((), ())),
                         preferred_element_type=jnp.float32)
    cv = lax.dot_general(oh16, v.astype(jnp.bfloat16), (((0,), (0,)), ((), ())),
                         preferred_element_type=jnp.float32)
    z = jnp.sum(onehot, axis=0)[None, :]

    @pl.when(t == 0)
    def _init():
        ko_ref[0] = ck
        vo_ref[0] = cv
        z_ref[...] = z

    @pl.when(t > 0)
    def _acc():
        ko_ref[0] += ck
        vo_ref[0] += cv
        z_ref[...] += z

    @pl.when(t == n_t - 1)
    def _norm():
        zs = jnp.clip(z_ref[...], 1e-8, None)[0, :, None]
        ko_ref[0] = ko_ref[0] / zs
        vo_ref[0] = vo_ref[0] / zs


def kernel(K_cold, V_cold, anchors):
    T, H, D = K_cold.shape
    G, M, _ = anchors.shape
    TB = min(8192, T)
    n_t = T // TB

    Kg = jnp.transpose(K_cold, (1, 0, 2))
    Vg = jnp.transpose(V_cold, (1, 0, 2))

    grid = (G, n_t)
    out_shape = [
        jax.ShapeDtypeStruct((G, M, D), jnp.float32),
        jax.ShapeDtypeStruct((G, M, D), jnp.float32),
    ]
    k_acc, v_acc = pl.pallas_call(
        functools.partial(_compactor_body, n_t=n_t),
        grid=grid,
        in_specs=[
            pl.BlockSpec((1, TB, D), lambda g, t: (g, t, 0)),
            pl.BlockSpec((1, TB, D), lambda g, t: (g, t, 0)),
            pl.BlockSpec((1, M, D), lambda g, t: (g, 0, 0)),
        ],
        out_specs=[
            pl.BlockSpec((1, M, D), lambda g, t: (g, 0, 0)),
            pl.BlockSpec((1, M, D), lambda g, t: (g, 0, 0)),
        ],
        scratch_shapes=[pltpu.VMEM((1, M), jnp.float32)],
        out_shape=out_shape,
    )(Kg, Vg, anchors)

    K_mem = jnp.transpose(k_acc, (1, 0, 2)).astype(K_cold.dtype)
    V_mem = jnp.transpose(v_acc, (1, 0, 2)).astype(V_cold.dtype)
    return (K_mem, V_mem)
